# baseline (speedup 1.0000x reference)
import functools

import jax
import jax.numpy as jnp
from jax import lax
from jax.experimental import pallas as pl
from jax.experimental.pallas import tpu as pltpu
from jax.experimental.pallas import tpu_sc as plsc

VOCAB_ = 1000000
DIM_ = 64
B_ = 16384
NNEG_ = 5
MARGIN_ = 1.0

NC = 2
NS = 16
NW = NC * NS
LANES = 16

BPW = B_ // NW
CHUNK = 128
NCHUNK = BPW // CHUNK
GROUPS = CHUNK // LANES


def _shuf(x, perm):
  return lax.gather(
      x, perm[:, None],
      lax.GatherDimensionNumbers(offset_dims=(), collapsed_slice_dims=(0,),
                                 start_index_map=(0,)),
      slice_sizes=(1,), mode=lax.GatherScatterMode.PROMISE_IN_BOUNDS)


def _sc_body(u_hbm, v_hbm, n0_hbm, n1_hbm, n2_hbm, n3_hbm, n4_hbm,
             src_hbm, tgt_hbm, out_hbm,
             iu, iv, in0, in1, in2, in3, in4,
             ru, rv, rn0, rn1, rn2, rn3, rn4,
             acc_v, sem):
  cid = lax.axis_index("c")
  sid = lax.axis_index("s")
  wid = cid * NS + sid

  lane = lax.iota(jnp.int32, LANES)
  perms = [lane ^ 1, lane ^ 2, lane ^ 4, lane ^ 8]
  total = jnp.zeros((LANES,), jnp.float32)

  for chunk in range(NCHUNK):
    base = wid * BPW + chunk * CHUNK
    nbase = base
    pltpu.sync_copy(u_hbm.at[pl.ds(base, CHUNK)], iu)
    pltpu.sync_copy(v_hbm.at[pl.ds(base, CHUNK)], iv)
    pltpu.sync_copy(n0_hbm.at[pl.ds(nbase, CHUNK)], in0)
    pltpu.sync_copy(n1_hbm.at[pl.ds(nbase, CHUNK)], in1)
    pltpu.sync_copy(n2_hbm.at[pl.ds(nbase, CHUNK)], in2)
    pltpu.sync_copy(n3_hbm.at[pl.ds(nbase, CHUNK)], in3)
    pltpu.sync_copy(n4_hbm.at[pl.ds(nbase, CHUNK)], in4)

    cps = [
        pltpu.make_async_copy(src_hbm.at[iu], ru, sem),
        pltpu.make_async_copy(tgt_hbm.at[iv], rv, sem),
        pltpu.make_async_copy(tgt_hbm.at[in0], rn0, sem),
        pltpu.make_async_copy(tgt_hbm.at[in1], rn1, sem),
        pltpu.make_async_copy(tgt_hbm.at[in2], rn2, sem),
        pltpu.make_async_copy(tgt_hbm.at[in3], rn3, sem),
        pltpu.make_async_copy(tgt_hbm.at[in4], rn4, sem),
    ]
    for cp in cps:
      cp.start()
    for cp in cps:
      cp.wait()

    def elem_body(e, tot):
      pv = jnp.zeros((LANES,), jnp.float32)
      p0 = jnp.zeros((LANES,), jnp.float32)
      p1 = jnp.zeros((LANES,), jnp.float32)
      p2 = jnp.zeros((LANES,), jnp.float32)
      p3 = jnp.zeros((LANES,), jnp.float32)
      p4 = jnp.zeros((LANES,), jnp.float32)
      for k in range(DIM_ // LANES):
        sl = pl.ds(k * LANES, LANES)
        uc = ru[e, sl]
        pv = pv + uc * rv[e, sl]
        p0 = p0 + uc * rn0[e, sl]
        p1 = p1 + uc * rn1[e, sl]
        p2 = p2 + uc * rn2[e, sl]
        p3 = p3 + uc * rn3[e, sl]
        p4 = p4 + uc * rn4[e, sl]
      loss = jnp.zeros((LANES,), jnp.float32)
      for p in (p0, p1, p2, p3, p4):
        r = p - pv
        for perm in perms:
          r = r + _shuf(r, perm)
        loss = loss + jnp.maximum(r + MARGIN_, 0.0)
      return tot + loss

    total = total + lax.fori_loop(0, CHUNK, elem_body,
                                  jnp.zeros((LANES,), jnp.float32))

  acc_v[...] = jnp.where(lane == 0, total, jnp.float32(0.0))
  pltpu.sync_copy(acc_v, out_hbm.at[wid])


@jax.jit
def _sc_call(u_pos, v_pos, n0, n1, n2, n3, n4, src_w, tgt_w):
  mesh = plsc.VectorSubcoreMesh(core_axis_name="c", subcore_axis_name="s")
  f = pl.kernel(
      _sc_body,
      out_type=jax.ShapeDtypeStruct((NW, LANES), jnp.float32),
      mesh=mesh,
      compiler_params=pltpu.CompilerParams(use_tc_tiling_on_sc=False),
      scratch_types=[
          pltpu.VMEM((CHUNK,), jnp.int32),
          pltpu.VMEM((CHUNK,), jnp.int32),
          pltpu.VMEM((CHUNK,), jnp.int32),
          pltpu.VMEM((CHUNK,), jnp.int32),
          pltpu.VMEM((CHUNK,), jnp.int32),
          pltpu.VMEM((CHUNK,), jnp.int32),
          pltpu.VMEM((CHUNK,), jnp.int32),
          pltpu.VMEM((CHUNK, DIM_), jnp.float32),
          pltpu.VMEM((CHUNK, DIM_), jnp.float32),
          pltpu.VMEM((CHUNK, DIM_), jnp.float32),
          pltpu.VMEM((CHUNK, DIM_), jnp.float32),
          pltpu.VMEM((CHUNK, DIM_), jnp.float32),
          pltpu.VMEM((CHUNK, DIM_), jnp.float32),
          pltpu.VMEM((CHUNK, DIM_), jnp.float32),
          pltpu.VMEM((LANES,), jnp.float32),
          pltpu.SemaphoreType.DMA,
      ],
  )
  return f(u_pos, v_pos, n0, n1, n2, n3, n4, src_w, tgt_w)


def kernel(u_pos, v_pos, v_neg, src_w, tgt_w):
  u_pos = u_pos.astype(jnp.int32)
  v_pos = v_pos.astype(jnp.int32)
  v_neg_t = v_neg.astype(jnp.int32).T
  partials = _sc_call(u_pos, v_pos,
                      v_neg_t[0], v_neg_t[1], v_neg_t[2], v_neg_t[3],
                      v_neg_t[4], src_w, tgt_w)
  return partials.sum() / jnp.float32(B_ * NNEG_)


# speedup vs baseline: 1.5212x; 1.5212x over previous
import functools

import jax
import jax.numpy as jnp
from jax import lax
from jax.experimental import pallas as pl
from jax.experimental.pallas import tpu as pltpu
from jax.experimental.pallas import tpu_sc as plsc

VOCAB_ = 1000000
DIM_ = 64
B_ = 16384
NNEG_ = 5
MARGIN_ = 1.0

NC = 2
NS = 16
NW = NC * NS
LANES = 16

BPW = B_ // NW
CHUNK = 128
NCHUNK = BPW // CHUNK
GROUPS = CHUNK // LANES


def _shuf(x, perm):
  return lax.gather(
      x, perm[:, None],
      lax.GatherDimensionNumbers(offset_dims=(), collapsed_slice_dims=(0,),
                                 start_index_map=(0,)),
      slice_sizes=(1,), mode=lax.GatherScatterMode.PROMISE_IN_BOUNDS)


def _sc_body(u_hbm, v_hbm, n0_hbm, n1_hbm, n2_hbm, n3_hbm, n4_hbm,
             src_hbm, tgt_hbm, out_hbm,
             iu, iv, in0, in1, in2, in3, in4,
             ru, rv, rn0, rn1, rn2, rn3, rn4,
             acc_v, sem):
  cid = lax.axis_index("c")
  sid = lax.axis_index("s")
  wid = cid * NS + sid

  lane = lax.iota(jnp.int32, LANES)
  perms = [lane ^ 1, lane ^ 2, lane ^ 4, lane ^ 8]
  total = jnp.zeros((LANES,), jnp.float32)

  for chunk in range(NCHUNK):
    base = wid * BPW + chunk * CHUNK
    nbase = base
    pltpu.sync_copy(u_hbm.at[pl.ds(base, CHUNK)], iu)
    pltpu.sync_copy(v_hbm.at[pl.ds(base, CHUNK)], iv)
    pltpu.sync_copy(n0_hbm.at[pl.ds(nbase, CHUNK)], in0)
    pltpu.sync_copy(n1_hbm.at[pl.ds(nbase, CHUNK)], in1)
    pltpu.sync_copy(n2_hbm.at[pl.ds(nbase, CHUNK)], in2)
    pltpu.sync_copy(n3_hbm.at[pl.ds(nbase, CHUNK)], in3)
    pltpu.sync_copy(n4_hbm.at[pl.ds(nbase, CHUNK)], in4)

    def row_dma(g, carry):
      gbase = g * LANES
      sl = pl.ds(gbase, LANES)
      vu, vv = iu[sl], iv[sl]
      v0, v1, v2, v3, v4 = in0[sl], in1[sl], in2[sl], in3[sl], in4[sl]
      for j in range(LANES):
        e = gbase + j
        pltpu.make_async_copy(src_hbm.at[vu[j]], ru.at[e], sem).start()
        pltpu.make_async_copy(tgt_hbm.at[vv[j]], rv.at[e], sem).start()
        pltpu.make_async_copy(tgt_hbm.at[v0[j]], rn0.at[e], sem).start()
        pltpu.make_async_copy(tgt_hbm.at[v1[j]], rn1.at[e], sem).start()
        pltpu.make_async_copy(tgt_hbm.at[v2[j]], rn2.at[e], sem).start()
        pltpu.make_async_copy(tgt_hbm.at[v3[j]], rn3.at[e], sem).start()
        pltpu.make_async_copy(tgt_hbm.at[v4[j]], rn4.at[e], sem).start()
      return carry

    lax.fori_loop(0, GROUPS, row_dma, jnp.int32(0))
    for buf in (ru, rv, rn0, rn1, rn2, rn3, rn4):
      pltpu.make_async_copy(src_hbm.at[pl.ds(0, CHUNK)], buf, sem).wait()

    def elem_body(e, tot):
      pv = jnp.zeros((LANES,), jnp.float32)
      p0 = jnp.zeros((LANES,), jnp.float32)
      p1 = jnp.zeros((LANES,), jnp.float32)
      p2 = jnp.zeros((LANES,), jnp.float32)
      p3 = jnp.zeros((LANES,), jnp.float32)
      p4 = jnp.zeros((LANES,), jnp.float32)
      for k in range(DIM_ // LANES):
        sl = pl.ds(k * LANES, LANES)
        uc = ru[e, sl]
        pv = pv + uc * rv[e, sl]
        p0 = p0 + uc * rn0[e, sl]
        p1 = p1 + uc * rn1[e, sl]
        p2 = p2 + uc * rn2[e, sl]
        p3 = p3 + uc * rn3[e, sl]
        p4 = p4 + uc * rn4[e, sl]
      loss = jnp.zeros((LANES,), jnp.float32)
      for p in (p0, p1, p2, p3, p4):
        r = p - pv
        for perm in perms:
          r = r + _shuf(r, perm)
        loss = loss + jnp.maximum(r + MARGIN_, 0.0)
      return tot + loss

    total = total + lax.fori_loop(0, CHUNK, elem_body,
                                  jnp.zeros((LANES,), jnp.float32))

  acc_v[...] = jnp.where(lane == 0, total, jnp.float32(0.0))
  pltpu.sync_copy(acc_v, out_hbm.at[pl.ds(wid * LANES, LANES)])


@jax.jit
def _sc_call(u_pos, v_pos, n0, n1, n2, n3, n4, src_w, tgt_w):
  mesh = plsc.VectorSubcoreMesh(core_axis_name="c", subcore_axis_name="s")
  f = pl.kernel(
      _sc_body,
      out_type=jax.ShapeDtypeStruct((NW * LANES,), jnp.float32),
      mesh=mesh,
      scratch_types=[
          pltpu.VMEM((CHUNK,), jnp.int32),
          pltpu.VMEM((CHUNK,), jnp.int32),
          pltpu.VMEM((CHUNK,), jnp.int32),
          pltpu.VMEM((CHUNK,), jnp.int32),
          pltpu.VMEM((CHUNK,), jnp.int32),
          pltpu.VMEM((CHUNK,), jnp.int32),
          pltpu.VMEM((CHUNK,), jnp.int32),
          pltpu.VMEM((CHUNK, DIM_), jnp.float32),
          pltpu.VMEM((CHUNK, DIM_), jnp.float32),
          pltpu.VMEM((CHUNK, DIM_), jnp.float32),
          pltpu.VMEM((CHUNK, DIM_), jnp.float32),
          pltpu.VMEM((CHUNK, DIM_), jnp.float32),
          pltpu.VMEM((CHUNK, DIM_), jnp.float32),
          pltpu.VMEM((CHUNK, DIM_), jnp.float32),
          pltpu.VMEM((LANES,), jnp.float32),
          pltpu.SemaphoreType.DMA,
      ],
  )
  return f(u_pos, v_pos, n0, n1, n2, n3, n4, src_w, tgt_w)


def kernel(u_pos, v_pos, v_neg, src_w, tgt_w):
  u_pos = u_pos.astype(jnp.int32)
  v_pos = v_pos.astype(jnp.int32)
  v_neg_t = v_neg.astype(jnp.int32).T
  partials = _sc_call(u_pos, v_pos,
                      v_neg_t[0], v_neg_t[1], v_neg_t[2], v_neg_t[3],
                      v_neg_t[4], src_w, tgt_w)
  return partials.sum() / jnp.float32(B_ * NNEG_)
